# baseline (speedup 1.0000x reference)
import jax
import jax.numpy as jnp
from jax.experimental import pallas as pl


def _matmul_block(x_ref, e_ref, o_ref):
    x = x_ref[...].astype(jnp.float32)
    o_ref[...] = jnp.dot(x, e_ref[...], preferred_element_type=jnp.float32)


def kernel(inputs, embeddings):
    M, K = inputs.shape
    _, N = embeddings.shape
    BM = 512
    out = pl.pallas_call(
        _matmul_block,
        grid=(M // BM,),
        in_specs=[
            pl.BlockSpec((BM, K), lambda i: (i, 0)),
            pl.BlockSpec((K, N), lambda i: (0, 0)),
        ],
        out_specs=pl.BlockSpec((BM, N), lambda i: (i, 0)),
        out_shape=jax.ShapeDtypeStruct((M, N), jnp.float32),
    )(inputs, embeddings)
    return out[1:]


# speedup vs baseline: 1.1283x; 1.1283x over previous
import jax
import jax.numpy as jnp
from jax.experimental import pallas as pl
from jax.experimental.pallas import tpu as pltpu


def _matmul_block(x_ref, e_ref, o_ref):
    x = x_ref[...].astype(jnp.float32)
    o_ref[...] = jnp.dot(x, e_ref[...], preferred_element_type=jnp.float32)


def kernel(inputs, embeddings):
    M, K = inputs.shape
    _, N = embeddings.shape
    BM = 2048
    out = pl.pallas_call(
        _matmul_block,
        grid=(M // BM,),
        in_specs=[
            pl.BlockSpec((BM, K), lambda i: (i, 0)),
            pl.BlockSpec(memory_space=pltpu.MemorySpace.VMEM),
        ],
        out_specs=pl.BlockSpec((BM, N), lambda i: (i, 0)),
        out_shape=jax.ShapeDtypeStruct((M, N), jnp.float32),
    )(inputs, embeddings)
    return out[1:]


# speedup vs baseline: 3.3524x; 2.9711x over previous
import jax
import jax.numpy as jnp
from jax.experimental import pallas as pl
from jax.experimental.pallas import tpu as pltpu


def _body(xt_ref, et_ref, o_ref, e_ref, prev_ref):
    i = pl.program_id(0)

    @pl.when(i == 0)
    def _():
        e_ref[...] = et_ref[...].T

    x = xt_ref[...].astype(jnp.float32)
    prod = jax.lax.dot_general(
        x, e_ref[...], (((0,), (0,)), ((), ())),
        preferred_element_type=jnp.float32,
    )
    carry = prev_ref[...]
    o_ref[...] = jnp.concatenate([prod[1:, :], carry], axis=0)
    prev_ref[...] = prod[0:1, :]


def kernel(inputs, embeddings):
    M, K = inputs.shape
    _, N = embeddings.shape
    xt = inputs.T
    et = embeddings.T
    BN = 2048
    nblk = M // BN
    return pl.pallas_call(
        _body,
        grid=(nblk,),
        in_specs=[
            pl.BlockSpec((K, BN), lambda i, n=nblk: (0, n - 1 - i)),
            pl.BlockSpec((N, K), lambda i: (0, 0)),
        ],
        out_specs=pl.BlockSpec((BN, N), lambda i, n=nblk: (n - 1 - i, 0)),
        out_shape=jax.ShapeDtypeStruct((M - 1, N), jnp.float32),
        scratch_shapes=[
            pltpu.VMEM((K, N), jnp.float32),
            pltpu.VMEM((1, N), jnp.float32),
        ],
    )(xt, et)
